# baseline (speedup 1.0000x reference)
import functools

import jax
import jax.numpy as jnp
from jax import lax
from jax.experimental import pallas as pl
from jax.experimental.pallas import tpu as pltpu
from jax.experimental.pallas import tpu_sc as plsc

NP_PAD = 10240
NW = 32
EB = 128
ROWS_PER_TILE = NP_PAD // 16

_MESH = plsc.VectorSubcoreMesh(core_axis_name="c", subcore_axis_name="s")


def _sc_degree(dst_blocks):
    nb = dst_blocks.shape[1]

    @functools.partial(
        pl.kernel,
        out_type=jax.ShapeDtypeStruct((2, NP_PAD, 16), jnp.float32),
        mesh=_MESH,
        scratch_types=[
            pltpu.VMEM((nb, EB), jnp.int32),
            pltpu.VMEM((EB, 16), jnp.float32),
            pltpu.VMEM((16, 16), jnp.float32),
            pltpu.VMEM_SHARED((NP_PAD, 16), jnp.float32),
        ],
    )
    def deg_kernel(dst_hbm, out_hbm, dst_v, ones_v, zb_v, acc):
        c = lax.axis_index("c")
        s = lax.axis_index("s")
        wid = c * 16 + s
        row0 = s * ROWS_PER_TILE
        zeros16 = jnp.zeros((16,), jnp.float32)
        ones16 = jnp.ones((16,), jnp.float32)
        for r in range(16):
            zb_v[r, :] = zeros16
        for r in range(EB):
            ones_v[r, :] = ones16

        @pl.loop(0, ROWS_PER_TILE // 16)
        def _(i):
            pltpu.sync_copy(zb_v, acc.at[pl.ds(row0 + i * 16, 16), :])

        pltpu.sync_copy(dst_hbm.at[wid], dst_v)
        plsc.subcore_barrier()

        @pl.loop(0, nb)
        def _(j):
            pltpu.sync_copy(ones_v, acc.at[dst_v.at[j]], add=True)

        plsc.subcore_barrier()

        @pl.loop(0, ROWS_PER_TILE // EB)
        def _(k):
            r = row0 + k * EB
            pltpu.sync_copy(acc.at[pl.ds(r, EB), :], ones_v)
            pltpu.sync_copy(ones_v, out_hbm.at[c, pl.ds(r, EB), :])

    return deg_kernel(dst_blocks)


def _sc_spmm(table, src_blocks, dst_blocks, feat):
    nb = src_blocks.shape[1]
    eb = src_blocks.shape[2]
    params = None
    if feat % 128 != 0:
        params = pltpu.CompilerParams(use_tc_tiling_on_sc=False)

    @functools.partial(
        pl.kernel,
        out_type=jax.ShapeDtypeStruct((2, NP_PAD, feat), jnp.float32),
        mesh=_MESH,
        compiler_params=params,
        scratch_types=[
            pltpu.VMEM((nb, eb), jnp.int32),
            pltpu.VMEM((nb, eb), jnp.int32),
            pltpu.VMEM((eb, feat), jnp.float32),
            pltpu.VMEM((eb, feat), jnp.float32),
            pltpu.VMEM((16, feat), jnp.float32),
            pltpu.VMEM_SHARED((NP_PAD, feat), jnp.float32),
            pltpu.SemaphoreType.DMA,
            pltpu.SemaphoreType.DMA,
            pltpu.SemaphoreType.DMA,
        ],
    )
    def spmm_kernel(tab_hbm, src_hbm, dst_hbm, out_hbm,
                    src_v, dst_v, buf0, buf1, zb_v, acc, gsem0, gsem1, zsem):
        c = lax.axis_index("c")
        s = lax.axis_index("s")
        wid = c * 16 + s
        row0 = s * ROWS_PER_TILE
        zeros16 = jnp.zeros((16,), jnp.float32)
        for r in range(16):
            for k in range(feat // 16):
                zb_v[r, pl.ds(k * 16, 16)] = zeros16

        nz = ROWS_PER_TILE // 16

        @pl.loop(0, nz)
        def _(i):
            pltpu.sync_copy(zb_v, acc.at[pl.ds(row0 + i * 16, 16), :])

        pltpu.sync_copy(src_hbm.at[wid], src_v)
        pltpu.sync_copy(dst_hbm.at[wid], dst_v)
        plsc.subcore_barrier()

        @pl.loop(0, nb)
        def _(j):
            pltpu.async_copy(tab_hbm.at[src_v.at[j]], buf0, gsem0).wait()
            pltpu.sync_copy(buf0, acc.at[dst_v.at[j]], add=True)

        plsc.subcore_barrier()

        @pl.loop(0, ROWS_PER_TILE // eb)
        def _(k):
            r = row0 + k * eb
            pltpu.sync_copy(acc.at[pl.ds(r, eb), :], buf0)
            pltpu.sync_copy(buf0, out_hbm.at[c, pl.ds(r, eb), :])

    return spmm_kernel(table, src_blocks, dst_blocks)


def _tc_scale(x_p, W1, dp0, dp1):
    rb = 1024
    d = x_p.shape[1]
    h = W1.shape[1]

    def body(x_ref, w_ref, d0_ref, d1_ref, xsa_ref, xsb_ref, a_ref):
        deg = d0_ref[...] + d1_ref[...] + 1.0
        a = lax.rsqrt(deg)
        xw = jnp.dot(x_ref[...], w_ref[...], preferred_element_type=jnp.float32)
        xs = xw * a
        xsa_ref[...] = xs[:, :h // 2]
        xsb_ref[...] = xs[:, h // 2:]
        a_ref[...] = a

    return pl.pallas_call(
        body,
        grid=(NP_PAD // rb,),
        in_specs=[
            pl.BlockSpec((rb, d), lambda i: (i, 0)),
            pl.BlockSpec((d, h), lambda i: (0, 0)),
            pl.BlockSpec((rb, 1), lambda i: (i, 0)),
            pl.BlockSpec((rb, 1), lambda i: (i, 0)),
        ],
        out_specs=[
            pl.BlockSpec((rb, h // 2), lambda i: (i, 0)),
            pl.BlockSpec((rb, h // 2), lambda i: (i, 0)),
            pl.BlockSpec((rb, 1), lambda i: (i, 0)),
        ],
        out_shape=[
            jax.ShapeDtypeStruct((NP_PAD, h // 2), jnp.float32),
            jax.ShapeDtypeStruct((NP_PAD, h // 2), jnp.float32),
            jax.ShapeDtypeStruct((NP_PAD, 1), jnp.float32),
        ],
    )(x_p, W1, dp0, dp1)


def _tc_hidden(p0a, p1a, p0b, p1b, xsa, xsb, a, b1, W2):
    rb = 1024
    hh = xsa.shape[1]
    h = 2 * hh
    co = W2.shape[1]

    def body(p0a_ref, p1a_ref, p0b_ref, p1b_ref, xsa_ref, xsb_ref,
             a_ref, b1_ref, w2_ref, out_ref):
        av = a_ref[...]
        b1v = b1_ref[...]
        w2 = w2_ref[...]
        ha = jnp.maximum(
            av * (p0a_ref[...] + p1a_ref[...] + xsa_ref[...]) + b1v[:, :hh], 0.0)
        hb = jnp.maximum(
            av * (p0b_ref[...] + p1b_ref[...] + xsb_ref[...]) + b1v[:, hh:], 0.0)
        hw = (jnp.dot(ha, w2[:hh, :], preferred_element_type=jnp.float32)
              + jnp.dot(hb, w2[hh:, :], preferred_element_type=jnp.float32))
        out_ref[...] = av * hw

    half = pl.BlockSpec((rb, hh), lambda i: (i, 0))
    return pl.pallas_call(
        body,
        grid=(NP_PAD // rb,),
        in_specs=[
            half, half, half, half, half, half,
            pl.BlockSpec((rb, 1), lambda i: (i, 0)),
            pl.BlockSpec((1, h), lambda i: (0, 0)),
            pl.BlockSpec((h, co), lambda i: (0, 0)),
        ],
        out_specs=pl.BlockSpec((rb, co), lambda i: (i, 0)),
        out_shape=jax.ShapeDtypeStruct((NP_PAD, co), jnp.float32),
    )(p0a, p1a, p0b, p1b, xsa, xsb, a, b1, W2)


def _tc_softmax(q0, q1, h2s, a, b2):
    rb = 1024
    co = h2s.shape[1]

    def body(q0_ref, q1_ref, h_ref, a_ref, b2_ref, out_ref):
        z = a_ref[...] * (q0_ref[...] + q1_ref[...] + h_ref[...]) + b2_ref[...]
        z = z - jnp.max(z, axis=1, keepdims=True)
        e = jnp.exp(z)
        out_ref[...] = e / jnp.sum(e, axis=1, keepdims=True)

    return pl.pallas_call(
        body,
        grid=(NP_PAD // rb,),
        in_specs=[
            pl.BlockSpec((rb, co), lambda i: (i, 0)),
            pl.BlockSpec((rb, co), lambda i: (i, 0)),
            pl.BlockSpec((rb, co), lambda i: (i, 0)),
            pl.BlockSpec((rb, 1), lambda i: (i, 0)),
            pl.BlockSpec((1, co), lambda i: (0, 0)),
        ],
        out_specs=pl.BlockSpec((rb, co), lambda i: (i, 0)),
        out_shape=jax.ShapeDtypeStruct((NP_PAD, co), jnp.float32),
    )(q0, q1, h2s, a, b2)


def kernel(node_embeddings, adjacency_lists, W1, b1, W2, b2):
    n, d = node_embeddings.shape
    e = adjacency_lists.shape[1]
    src = adjacency_lists[0].astype(jnp.int32)
    dst = adjacency_lists[1].astype(jnp.int32)

    nb = -(-e // (NW * EB))
    nb += nb % 2
    ep = NW * nb * EB
    src_flat = jnp.concatenate([src, jnp.zeros((ep - e,), jnp.int32)])
    dst_flat = jnp.concatenate([dst, jnp.full((ep - e,), n, jnp.int32)])
    src_p = src_flat.reshape(NW, nb, EB)
    dst_p = dst_flat.reshape(NW, nb, EB)
    x_p = jnp.pad(node_embeddings, ((0, NP_PAD - n), (0, 0)))

    degp = _sc_degree(dst_p)
    dp0 = degp[0, :, :1]
    dp1 = degp[1, :, :1]
    xsa, xsb, a = _tc_scale(x_p, W1, dp0, dp1)
    agg1a = _sc_spmm(xsa, src_p, dst_p, W1.shape[1] // 2)
    agg1b = _sc_spmm(xsb, src_p, dst_p, W1.shape[1] // 2)
    h2s = _tc_hidden(agg1a[0], agg1a[1], agg1b[0], agg1b[1], xsa, xsb, a,
                     b1.reshape(1, -1), W2)
    agg2 = _sc_spmm(h2s, src_p, dst_p, W2.shape[1])
    probs = _tc_softmax(agg2[0], agg2[1], h2s, a, b2.reshape(1, -1))
    return probs[:n]


# speedup vs baseline: 1.0913x; 1.0913x over previous
import functools

import jax
import jax.numpy as jnp
from jax import lax
from jax.experimental import pallas as pl
from jax.experimental.pallas import tpu as pltpu
from jax.experimental.pallas import tpu_sc as plsc

NP_PAD = 10240
NW = 32
EB = 128
ROWS_PER_TILE = NP_PAD // 16

_MESH = plsc.VectorSubcoreMesh(core_axis_name="c", subcore_axis_name="s")


def _sc_degree(dst_blocks):
    nb = dst_blocks.shape[1]

    @functools.partial(
        pl.kernel,
        out_type=jax.ShapeDtypeStruct((2, NP_PAD, 16), jnp.float32),
        mesh=_MESH,
        scratch_types=[
            pltpu.VMEM((nb, EB), jnp.int32),
            pltpu.VMEM((EB, 16), jnp.float32),
            pltpu.VMEM((16, 16), jnp.float32),
            pltpu.VMEM_SHARED((NP_PAD, 16), jnp.float32),
        ],
    )
    def deg_kernel(dst_hbm, out_hbm, dst_v, ones_v, zb_v, acc):
        c = lax.axis_index("c")
        s = lax.axis_index("s")
        wid = c * 16 + s
        row0 = s * ROWS_PER_TILE
        zeros16 = jnp.zeros((16,), jnp.float32)
        ones16 = jnp.ones((16,), jnp.float32)
        for r in range(16):
            zb_v[r, :] = zeros16
        for r in range(EB):
            ones_v[r, :] = ones16

        @pl.loop(0, ROWS_PER_TILE // 16)
        def _(i):
            pltpu.sync_copy(zb_v, acc.at[pl.ds(row0 + i * 16, 16), :])

        pltpu.sync_copy(dst_hbm.at[wid], dst_v)
        plsc.subcore_barrier()

        @pl.loop(0, nb)
        def _(j):
            pltpu.sync_copy(ones_v, acc.at[dst_v.at[j]], add=True)

        plsc.subcore_barrier()

        @pl.loop(0, ROWS_PER_TILE // EB)
        def _(k):
            r = row0 + k * EB
            pltpu.sync_copy(acc.at[pl.ds(r, EB), :], ones_v)
            pltpu.sync_copy(ones_v, out_hbm.at[c, pl.ds(r, EB), :])

    return deg_kernel(dst_blocks)


def _sc_spmm(table, src_blocks, dst_blocks, feat):
    nb = src_blocks.shape[1]
    eb = src_blocks.shape[2]
    params = None
    if feat % 128 != 0:
        params = pltpu.CompilerParams(use_tc_tiling_on_sc=False)

    sb = 8
    assert nb % sb == 0

    @functools.partial(
        pl.kernel,
        out_type=jax.ShapeDtypeStruct((2, NP_PAD, feat), jnp.float32),
        mesh=_MESH,
        compiler_params=params,
        scratch_types=[
            pltpu.VMEM((nb, eb), jnp.int32),
            pltpu.VMEM((nb, eb), jnp.int32),
            [pltpu.VMEM((eb, feat), jnp.float32) for _ in range(sb)],
            pltpu.VMEM((16, feat), jnp.float32),
            pltpu.VMEM_SHARED((NP_PAD, feat), jnp.float32),
            pltpu.SemaphoreType.DMA,
            pltpu.SemaphoreType.DMA,
            pltpu.SemaphoreType.DMA,
        ],
    )
    def spmm_kernel(tab_hbm, src_hbm, dst_hbm, out_hbm,
                    src_v, dst_v, bufs, zb_v, acc, gsem, ssem, zsem):
        c = lax.axis_index("c")
        s = lax.axis_index("s")
        wid = c * 16 + s
        row0 = s * ROWS_PER_TILE
        zeros16 = jnp.zeros((16,), jnp.float32)
        for r in range(16):
            for k in range(feat // 16):
                zb_v[r, pl.ds(k * 16, 16)] = zeros16

        nz = ROWS_PER_TILE // 16

        @pl.loop(0, nz)
        def _(i):
            pltpu.sync_copy(zb_v, acc.at[pl.ds(row0 + i * 16, 16), :])

        pltpu.sync_copy(src_hbm.at[wid], src_v)
        pltpu.sync_copy(dst_hbm.at[wid], dst_v)
        plsc.subcore_barrier()

        @pl.loop(0, nb // sb)
        def _(i):
            j0 = i * sb
            gds = [
                pltpu.async_copy(tab_hbm.at[src_v.at[j0 + b]], bufs[b], gsem)
                for b in range(sb)
            ]
            for d in gds:
                d.wait()
            for b in range(sb):
                pltpu.sync_copy(bufs[b], acc.at[dst_v.at[j0 + b]], add=True)

        plsc.subcore_barrier()

        @pl.loop(0, ROWS_PER_TILE // eb)
        def _(k):
            r = row0 + k * eb
            pltpu.sync_copy(acc.at[pl.ds(r, eb), :], bufs[0])
            pltpu.sync_copy(bufs[0], out_hbm.at[c, pl.ds(r, eb), :])

    return spmm_kernel(table, src_blocks, dst_blocks)


def _tc_scale(x_p, W1, dp0, dp1):
    rb = 1024
    d = x_p.shape[1]
    h = W1.shape[1]

    def body(x_ref, w_ref, d0_ref, d1_ref, xsa_ref, xsb_ref, a_ref):
        deg = d0_ref[...] + d1_ref[...] + 1.0
        a = lax.rsqrt(deg)
        xw = jnp.dot(x_ref[...], w_ref[...], preferred_element_type=jnp.float32)
        xs = xw * a
        xsa_ref[...] = xs[:, :h // 2]
        xsb_ref[...] = xs[:, h // 2:]
        a_ref[...] = a

    return pl.pallas_call(
        body,
        grid=(NP_PAD // rb,),
        in_specs=[
            pl.BlockSpec((rb, d), lambda i: (i, 0)),
            pl.BlockSpec((d, h), lambda i: (0, 0)),
            pl.BlockSpec((rb, 1), lambda i: (i, 0)),
            pl.BlockSpec((rb, 1), lambda i: (i, 0)),
        ],
        out_specs=[
            pl.BlockSpec((rb, h // 2), lambda i: (i, 0)),
            pl.BlockSpec((rb, h // 2), lambda i: (i, 0)),
            pl.BlockSpec((rb, 1), lambda i: (i, 0)),
        ],
        out_shape=[
            jax.ShapeDtypeStruct((NP_PAD, h // 2), jnp.float32),
            jax.ShapeDtypeStruct((NP_PAD, h // 2), jnp.float32),
            jax.ShapeDtypeStruct((NP_PAD, 1), jnp.float32),
        ],
    )(x_p, W1, dp0, dp1)


def _tc_hidden(p0a, p1a, p0b, p1b, xsa, xsb, a, b1, W2):
    rb = 1024
    hh = xsa.shape[1]
    h = 2 * hh
    co = W2.shape[1]

    def body(p0a_ref, p1a_ref, p0b_ref, p1b_ref, xsa_ref, xsb_ref,
             a_ref, b1_ref, w2_ref, out_ref):
        av = a_ref[...]
        b1v = b1_ref[...]
        w2 = w2_ref[...]
        ha = jnp.maximum(
            av * (p0a_ref[...] + p1a_ref[...] + xsa_ref[...]) + b1v[:, :hh], 0.0)
        hb = jnp.maximum(
            av * (p0b_ref[...] + p1b_ref[...] + xsb_ref[...]) + b1v[:, hh:], 0.0)
        hw = (jnp.dot(ha, w2[:hh, :], preferred_element_type=jnp.float32)
              + jnp.dot(hb, w2[hh:, :], preferred_element_type=jnp.float32))
        out_ref[...] = av * hw

    half = pl.BlockSpec((rb, hh), lambda i: (i, 0))
    return pl.pallas_call(
        body,
        grid=(NP_PAD // rb,),
        in_specs=[
            half, half, half, half, half, half,
            pl.BlockSpec((rb, 1), lambda i: (i, 0)),
            pl.BlockSpec((1, h), lambda i: (0, 0)),
            pl.BlockSpec((h, co), lambda i: (0, 0)),
        ],
        out_specs=pl.BlockSpec((rb, co), lambda i: (i, 0)),
        out_shape=jax.ShapeDtypeStruct((NP_PAD, co), jnp.float32),
    )(p0a, p1a, p0b, p1b, xsa, xsb, a, b1, W2)


def _tc_softmax(q0, q1, h2s, a, b2):
    rb = 1024
    co = h2s.shape[1]

    def body(q0_ref, q1_ref, h_ref, a_ref, b2_ref, out_ref):
        z = a_ref[...] * (q0_ref[...] + q1_ref[...] + h_ref[...]) + b2_ref[...]
        z = z - jnp.max(z, axis=1, keepdims=True)
        e = jnp.exp(z)
        out_ref[...] = e / jnp.sum(e, axis=1, keepdims=True)

    return pl.pallas_call(
        body,
        grid=(NP_PAD // rb,),
        in_specs=[
            pl.BlockSpec((rb, co), lambda i: (i, 0)),
            pl.BlockSpec((rb, co), lambda i: (i, 0)),
            pl.BlockSpec((rb, co), lambda i: (i, 0)),
            pl.BlockSpec((rb, 1), lambda i: (i, 0)),
            pl.BlockSpec((1, co), lambda i: (0, 0)),
        ],
        out_specs=pl.BlockSpec((rb, co), lambda i: (i, 0)),
        out_shape=jax.ShapeDtypeStruct((NP_PAD, co), jnp.float32),
    )(q0, q1, h2s, a, b2)


def kernel(node_embeddings, adjacency_lists, W1, b1, W2, b2):
    n, d = node_embeddings.shape
    e = adjacency_lists.shape[1]
    src = adjacency_lists[0].astype(jnp.int32)
    dst = adjacency_lists[1].astype(jnp.int32)

    nb = -(-e // (NW * EB))
    nb += nb % 2
    ep = NW * nb * EB
    src_flat = jnp.concatenate([src, jnp.zeros((ep - e,), jnp.int32)])
    dst_flat = jnp.concatenate([dst, jnp.full((ep - e,), n, jnp.int32)])
    src_p = src_flat.reshape(NW, nb, EB)
    dst_p = dst_flat.reshape(NW, nb, EB)
    x_p = jnp.pad(node_embeddings, ((0, NP_PAD - n), (0, 0)))

    degp = _sc_degree(dst_p)
    dp0 = degp[0, :, :1]
    dp1 = degp[1, :, :1]
    xsa, xsb, a = _tc_scale(x_p, W1, dp0, dp1)
    agg1a = _sc_spmm(xsa, src_p, dst_p, W1.shape[1] // 2)
    agg1b = _sc_spmm(xsb, src_p, dst_p, W1.shape[1] // 2)
    h2s = _tc_hidden(agg1a[0], agg1a[1], agg1b[0], agg1b[1], xsa, xsb, a,
                     b1.reshape(1, -1), W2)
    agg2 = _sc_spmm(h2s, src_p, dst_p, W2.shape[1])
    probs = _tc_softmax(agg2[0], agg2[1], h2s, a, b2.reshape(1, -1))
    return probs[:n]


# speedup vs baseline: 1.1001x; 1.0081x over previous
import functools

import jax
import jax.numpy as jnp
from jax import lax
from jax.experimental import pallas as pl
from jax.experimental.pallas import tpu as pltpu
from jax.experimental.pallas import tpu_sc as plsc

NP_PAD = 10240
NW = 32
EB = 128
ROWS_PER_TILE = NP_PAD // 16

_MESH = plsc.VectorSubcoreMesh(core_axis_name="c", subcore_axis_name="s")


def _sc_degree(dst_blocks):
    nb = dst_blocks.shape[1]

    @functools.partial(
        pl.kernel,
        out_type=jax.ShapeDtypeStruct((2, NP_PAD, 16), jnp.float32),
        mesh=_MESH,
        scratch_types=[
            pltpu.VMEM((nb, EB), jnp.int32),
            pltpu.VMEM((EB, 16), jnp.float32),
            pltpu.VMEM((16, 16), jnp.float32),
            pltpu.VMEM_SHARED((NP_PAD, 16), jnp.float32),
        ],
    )
    def deg_kernel(dst_hbm, out_hbm, dst_v, ones_v, zb_v, acc):
        c = lax.axis_index("c")
        s = lax.axis_index("s")
        wid = c * 16 + s
        row0 = s * ROWS_PER_TILE
        zeros16 = jnp.zeros((16,), jnp.float32)
        ones16 = jnp.ones((16,), jnp.float32)
        for r in range(16):
            zb_v[r, :] = zeros16
        for r in range(EB):
            ones_v[r, :] = ones16

        @pl.loop(0, ROWS_PER_TILE // 16)
        def _(i):
            pltpu.sync_copy(zb_v, acc.at[pl.ds(row0 + i * 16, 16), :])

        pltpu.sync_copy(dst_hbm.at[wid], dst_v)
        plsc.subcore_barrier()

        @pl.loop(0, nb)
        def _(j):
            pltpu.sync_copy(ones_v, acc.at[dst_v.at[j]], add=True)

        plsc.subcore_barrier()

        @pl.loop(0, ROWS_PER_TILE // EB)
        def _(k):
            r = row0 + k * EB
            pltpu.sync_copy(acc.at[pl.ds(r, EB), :], ones_v)
            pltpu.sync_copy(ones_v, out_hbm.at[c, pl.ds(r, EB), :])

    return deg_kernel(dst_blocks)


def _sc_spmm(table, src_blocks, dst_blocks, feat):
    nb = src_blocks.shape[1]
    eb = src_blocks.shape[2]
    params = None
    if feat % 128 != 0:
        params = pltpu.CompilerParams(use_tc_tiling_on_sc=False)

    sb = 8
    assert nb % sb == 0

    @functools.partial(
        pl.kernel,
        out_type=jax.ShapeDtypeStruct((2, NP_PAD, feat), jnp.float32),
        mesh=_MESH,
        compiler_params=params,
        scratch_types=[
            pltpu.VMEM((nb, eb), jnp.int32),
            pltpu.VMEM((nb, eb), jnp.int32),
            [pltpu.VMEM((eb, feat), jnp.float32) for _ in range(sb)],
            pltpu.VMEM((16, feat), jnp.float32),
            pltpu.VMEM_SHARED((NP_PAD, feat), jnp.float32),
            pltpu.SemaphoreType.DMA,
            pltpu.SemaphoreType.DMA,
            pltpu.SemaphoreType.DMA,
        ],
    )
    def spmm_kernel(tab_hbm, src_hbm, dst_hbm, out_hbm,
                    src_v, dst_v, bufs, zb_v, acc, gsem, ssem, zsem):
        c = lax.axis_index("c")
        s = lax.axis_index("s")
        wid = c * 16 + s
        row0 = s * ROWS_PER_TILE
        zeros16 = jnp.zeros((16,), jnp.float32)
        for r in range(16):
            for k in range(feat // 16):
                zb_v[r, pl.ds(k * 16, 16)] = zeros16

        nz = ROWS_PER_TILE // 16

        @pl.loop(0, nz)
        def _(i):
            pltpu.sync_copy(zb_v, acc.at[pl.ds(row0 + i * 16, 16), :])

        pltpu.sync_copy(src_hbm.at[wid], src_v)
        pltpu.sync_copy(dst_hbm.at[wid], dst_v)
        plsc.subcore_barrier()

        @pl.loop(0, nb // sb)
        def _(i):
            j0 = i * sb
            gds = [
                pltpu.async_copy(tab_hbm.at[src_v.at[j0 + b]], bufs[b], gsem)
                for b in range(sb)
            ]
            for d in gds:
                d.wait()
            sds = [
                pltpu.async_copy(bufs[b], acc.at[dst_v.at[j0 + b]], ssem,
                                 add=True)
                for b in range(sb)
            ]
            for d in sds:
                d.wait()

        plsc.subcore_barrier()

        @pl.loop(0, ROWS_PER_TILE // eb)
        def _(k):
            r = row0 + k * eb
            pltpu.sync_copy(acc.at[pl.ds(r, eb), :], bufs[0])
            pltpu.sync_copy(bufs[0], out_hbm.at[c, pl.ds(r, eb), :])

    return spmm_kernel(table, src_blocks, dst_blocks)


def _tc_scale(x_p, W1, dp0, dp1):
    rb = 1024
    d = x_p.shape[1]
    h = W1.shape[1]

    def body(x_ref, w_ref, d0_ref, d1_ref, xsa_ref, xsb_ref, a_ref):
        deg = d0_ref[...] + d1_ref[...] + 1.0
        a = lax.rsqrt(deg)
        xw = jnp.dot(x_ref[...], w_ref[...], preferred_element_type=jnp.float32)
        xs = xw * a
        xsa_ref[...] = xs[:, :h // 2]
        xsb_ref[...] = xs[:, h // 2:]
        a_ref[...] = a

    return pl.pallas_call(
        body,
        grid=(NP_PAD // rb,),
        in_specs=[
            pl.BlockSpec((rb, d), lambda i: (i, 0)),
            pl.BlockSpec((d, h), lambda i: (0, 0)),
            pl.BlockSpec((rb, 1), lambda i: (i, 0)),
            pl.BlockSpec((rb, 1), lambda i: (i, 0)),
        ],
        out_specs=[
            pl.BlockSpec((rb, h // 2), lambda i: (i, 0)),
            pl.BlockSpec((rb, h // 2), lambda i: (i, 0)),
            pl.BlockSpec((rb, 1), lambda i: (i, 0)),
        ],
        out_shape=[
            jax.ShapeDtypeStruct((NP_PAD, h // 2), jnp.float32),
            jax.ShapeDtypeStruct((NP_PAD, h // 2), jnp.float32),
            jax.ShapeDtypeStruct((NP_PAD, 1), jnp.float32),
        ],
    )(x_p, W1, dp0, dp1)


def _tc_hidden(p0a, p1a, p0b, p1b, xsa, xsb, a, b1, W2):
    rb = 1024
    hh = xsa.shape[1]
    h = 2 * hh
    co = W2.shape[1]

    def body(p0a_ref, p1a_ref, p0b_ref, p1b_ref, xsa_ref, xsb_ref,
             a_ref, b1_ref, w2_ref, out_ref):
        av = a_ref[...]
        b1v = b1_ref[...]
        w2 = w2_ref[...]
        ha = jnp.maximum(
            av * (p0a_ref[...] + p1a_ref[...] + xsa_ref[...]) + b1v[:, :hh], 0.0)
        hb = jnp.maximum(
            av * (p0b_ref[...] + p1b_ref[...] + xsb_ref[...]) + b1v[:, hh:], 0.0)
        hw = (jnp.dot(ha, w2[:hh, :], preferred_element_type=jnp.float32)
              + jnp.dot(hb, w2[hh:, :], preferred_element_type=jnp.float32))
        out_ref[...] = av * hw

    half = pl.BlockSpec((rb, hh), lambda i: (i, 0))
    return pl.pallas_call(
        body,
        grid=(NP_PAD // rb,),
        in_specs=[
            half, half, half, half, half, half,
            pl.BlockSpec((rb, 1), lambda i: (i, 0)),
            pl.BlockSpec((1, h), lambda i: (0, 0)),
            pl.BlockSpec((h, co), lambda i: (0, 0)),
        ],
        out_specs=pl.BlockSpec((rb, co), lambda i: (i, 0)),
        out_shape=jax.ShapeDtypeStruct((NP_PAD, co), jnp.float32),
    )(p0a, p1a, p0b, p1b, xsa, xsb, a, b1, W2)


def _tc_softmax(q0, q1, h2s, a, b2):
    rb = 1024
    co = h2s.shape[1]

    def body(q0_ref, q1_ref, h_ref, a_ref, b2_ref, out_ref):
        z = a_ref[...] * (q0_ref[...] + q1_ref[...] + h_ref[...]) + b2_ref[...]
        z = z - jnp.max(z, axis=1, keepdims=True)
        e = jnp.exp(z)
        out_ref[...] = e / jnp.sum(e, axis=1, keepdims=True)

    return pl.pallas_call(
        body,
        grid=(NP_PAD // rb,),
        in_specs=[
            pl.BlockSpec((rb, co), lambda i: (i, 0)),
            pl.BlockSpec((rb, co), lambda i: (i, 0)),
            pl.BlockSpec((rb, co), lambda i: (i, 0)),
            pl.BlockSpec((rb, 1), lambda i: (i, 0)),
            pl.BlockSpec((1, co), lambda i: (0, 0)),
        ],
        out_specs=pl.BlockSpec((rb, co), lambda i: (i, 0)),
        out_shape=jax.ShapeDtypeStruct((NP_PAD, co), jnp.float32),
    )(q0, q1, h2s, a, b2)


def kernel(node_embeddings, adjacency_lists, W1, b1, W2, b2):
    n, d = node_embeddings.shape
    e = adjacency_lists.shape[1]
    src = adjacency_lists[0].astype(jnp.int32)
    dst = adjacency_lists[1].astype(jnp.int32)

    nb = -(-e // (NW * EB))
    nb += nb % 2
    ep = NW * nb * EB
    src_flat = jnp.concatenate([src, jnp.zeros((ep - e,), jnp.int32)])
    dst_flat = jnp.concatenate([dst, jnp.full((ep - e,), n, jnp.int32)])
    src_p = src_flat.reshape(NW, nb, EB)
    dst_p = dst_flat.reshape(NW, nb, EB)
    x_p = jnp.pad(node_embeddings, ((0, NP_PAD - n), (0, 0)))

    degp = _sc_degree(dst_p)
    dp0 = degp[0, :, :1]
    dp1 = degp[1, :, :1]
    xsa, xsb, a = _tc_scale(x_p, W1, dp0, dp1)
    agg1a = _sc_spmm(xsa, src_p, dst_p, W1.shape[1] // 2)
    agg1b = _sc_spmm(xsb, src_p, dst_p, W1.shape[1] // 2)
    h2s = _tc_hidden(agg1a[0], agg1a[1], agg1b[0], agg1b[1], xsa, xsb, a,
                     b1.reshape(1, -1), W2)
    agg2 = _sc_spmm(h2s, src_p, dst_p, W2.shape[1])
    probs = _tc_softmax(agg2[0], agg2[1], h2s, a, b2.reshape(1, -1))
    return probs[:n]


# speedup vs baseline: 2.4739x; 2.2488x over previous
import functools

import jax
import jax.numpy as jnp
from jax import lax
from jax.experimental import pallas as pl
from jax.experimental.pallas import tpu as pltpu
from jax.experimental.pallas import tpu_sc as plsc

NP_PAD = 10240
NW = 32
EB = 128
ROWS_PER_TILE = NP_PAD // 16

_MESH = plsc.VectorSubcoreMesh(core_axis_name="c", subcore_axis_name="s")


def _zero_stores(ref, rows, feat):
    zeros16 = jnp.zeros((16,), jnp.float32)
    for r in range(rows):
        for k in range(feat // 16):
            ref[r, pl.ds(k * 16, 16)] = zeros16


def _zero_acc_batched(zb_v, acc, row0, zsem):
    nz = ROWS_PER_TILE // 16
    for g in range(nz // 8):
        ds = [
            pltpu.async_copy(
                zb_v, acc.at[pl.ds(row0 + (g * 8 + b) * 16, 16), :], zsem)
            for b in range(8)
        ]
        for d in ds:
            d.wait()


def _load_my_rows(rows_hbm, rows_v, wid, base, extra):
    pltpu.sync_copy(rows_hbm.at[pl.ds(wid * base, base), :],
                    rows_v.at[pl.ds(0, base), :])
    if extra:
        @pl.when(wid < extra)
        def _():
            pltpu.sync_copy(rows_hbm.at[pl.ds(base * NW + wid, 1), :],
                            rows_v.at[pl.ds(base, 1), :])


def _sc_degree(dst_rows):
    er = dst_rows.shape[0]
    base = er // NW
    extra = er % NW
    sb = 8

    @functools.partial(
        pl.kernel,
        out_type=jax.ShapeDtypeStruct((2, NP_PAD, 16), jnp.float32),
        mesh=_MESH,
        compiler_params=pltpu.CompilerParams(use_tc_tiling_on_sc=False),
        scratch_types=[
            pltpu.VMEM((base + 1, EB), jnp.int32),
            pltpu.VMEM((EB, 16), jnp.float32),
            pltpu.VMEM((16, 16), jnp.float32),
            pltpu.VMEM_SHARED((NP_PAD, 16), jnp.float32),
            pltpu.SemaphoreType.DMA,
            pltpu.SemaphoreType.DMA,
        ],
    )
    def deg_kernel(dst_hbm, out_hbm, dst_v, ones_v, zb_v, acc, ssem, zsem):
        c = lax.axis_index("c")
        s = lax.axis_index("s")
        wid = c * 16 + s
        row0 = s * ROWS_PER_TILE
        _zero_stores(zb_v, 16, 16)
        ones16 = jnp.ones((16,), jnp.float32)
        for r in range(EB):
            ones_v[r, :] = ones16
        _zero_acc_batched(zb_v, acc, row0, zsem)
        _load_my_rows(dst_hbm, dst_v, wid, base, extra)
        plsc.subcore_barrier()

        @pl.loop(0, base // sb)
        def _(i):
            j0 = i * sb
            ds = [
                pltpu.async_copy(ones_v, acc.at[dst_v.at[j0 + b]], ssem,
                                 add=True)
                for b in range(sb)
            ]
            for d in ds:
                d.wait()

        for j in range(base - base % sb, base):
            pltpu.sync_copy(ones_v, acc.at[dst_v.at[j]], add=True)
        if extra:
            @pl.when(wid < extra)
            def _():
                pltpu.sync_copy(ones_v, acc.at[dst_v.at[base]], add=True)
        plsc.subcore_barrier()

        @pl.loop(0, ROWS_PER_TILE // EB)
        def _(k):
            r = row0 + k * EB
            pltpu.sync_copy(acc.at[pl.ds(r, EB), :], ones_v)
            pltpu.sync_copy(ones_v, out_hbm.at[c, pl.ds(r, EB), :])

    return deg_kernel(dst_rows)


def _sc_spmm(table, src_rows, dst_rows, feat):
    er = src_rows.shape[0]
    base = er // NW
    extra = er % NW
    params = None
    if feat % 128 != 0:
        params = pltpu.CompilerParams(use_tc_tiling_on_sc=False)
    sb = 8 if feat <= 16 else 3

    @functools.partial(
        pl.kernel,
        out_type=jax.ShapeDtypeStruct((2, NP_PAD, feat), jnp.float32),
        mesh=_MESH,
        compiler_params=params,
        scratch_types=[
            pltpu.VMEM((base + 1, EB), jnp.int32),
            pltpu.VMEM((base + 1, EB), jnp.int32),
            [pltpu.VMEM((EB, feat), jnp.float32) for _ in range(sb)],
            pltpu.VMEM((16, feat), jnp.float32),
            pltpu.VMEM_SHARED((NP_PAD, feat), jnp.float32),
            pltpu.VMEM_SHARED((NP_PAD, feat), jnp.float32),
            pltpu.SemaphoreType.DMA,
            pltpu.SemaphoreType.DMA,
        ],
    )
    def spmm_kernel(tab_hbm, src_hbm, dst_hbm, out_hbm,
                    src_v, dst_v, bufs, zb_v, tab_s, acc, gsem, ssem):
        c = lax.axis_index("c")
        s = lax.axis_index("s")
        wid = c * 16 + s
        row0 = s * ROWS_PER_TILE
        _zero_stores(zb_v, 16, feat)

        @pl.loop(0, ROWS_PER_TILE // EB)
        def _(k):
            r = row0 + k * EB
            pltpu.async_copy(tab_hbm.at[pl.ds(r, EB), :], bufs[0], gsem).wait()
            pltpu.sync_copy(bufs[0], tab_s.at[pl.ds(r, EB), :])

        _zero_acc_batched(zb_v, acc, row0, ssem)
        _load_my_rows(src_hbm, src_v, wid, base, extra)
        _load_my_rows(dst_hbm, dst_v, wid, base, extra)
        plsc.subcore_barrier()

        def edge_batch(j0, nblk):
            gds = [
                pltpu.async_copy(tab_s.at[src_v.at[j0 + b]], bufs[b], gsem)
                for b in range(nblk)
            ]
            for d in gds:
                d.wait()
            sds = [
                pltpu.async_copy(bufs[b], acc.at[dst_v.at[j0 + b]], ssem,
                                 add=True)
                for b in range(nblk)
            ]
            for d in sds:
                d.wait()

        @pl.loop(0, base // sb)
        def _(i):
            edge_batch(i * sb, sb)

        if base % sb:
            edge_batch(base - base % sb, base % sb)
        if extra:
            @pl.when(wid < extra)
            def _():
                edge_batch(base, 1)
        plsc.subcore_barrier()

        @pl.loop(0, ROWS_PER_TILE // EB)
        def _(k):
            r = row0 + k * EB
            pltpu.sync_copy(acc.at[pl.ds(r, EB), :], bufs[0])
            pltpu.sync_copy(bufs[0], out_hbm.at[c, pl.ds(r, EB), :])

    return spmm_kernel(table, src_rows, dst_rows)


def _tc_scale(x, W1, degp):
    n, d = x.shape
    h = W1.shape[1]
    hh = h // 2
    rb = 1000

    def body(x_ref, w_ref, dp_ref, xsa_ref, xsb_ref, a_ref):
        deg = dp_ref[0, :, :1] + dp_ref[1, :, :1] + 1.0
        a = lax.rsqrt(deg)
        xw = jnp.dot(x_ref[...], w_ref[...], preferred_element_type=jnp.float32)
        xs = xw * a
        xsa_ref[...] = xs[:, :hh]
        xsb_ref[...] = xs[:, hh:]
        a_ref[...] = a

    return pl.pallas_call(
        body,
        grid=(n // rb,),
        in_specs=[
            pl.BlockSpec((rb, d), lambda i: (i, 0)),
            pl.BlockSpec((d, h), lambda i: (0, 0)),
            pl.BlockSpec((2, rb, 16), lambda i: (0, i, 0)),
        ],
        out_specs=[
            pl.BlockSpec((rb, hh), lambda i: (i, 0)),
            pl.BlockSpec((rb, hh), lambda i: (i, 0)),
            pl.BlockSpec((rb, 1), lambda i: (i, 0)),
        ],
        out_shape=[
            jax.ShapeDtypeStruct((NP_PAD, hh), jnp.float32),
            jax.ShapeDtypeStruct((NP_PAD, hh), jnp.float32),
            jax.ShapeDtypeStruct((NP_PAD, 1), jnp.float32),
        ],
    )(x, W1, degp)


def _tc_hidden(p1a, p1b, xsa, xsb, a, b1, W2):
    hh = xsa.shape[1]
    h = 2 * hh
    co = W2.shape[1]
    rb = 1000
    n = 10 * rb

    def body(p1a_ref, p1b_ref, xsa_ref, xsb_ref, a_ref, b1_ref, w2_ref,
             out_ref):
        av = a_ref[...]
        b1v = b1_ref[...]
        w2 = w2_ref[...]
        ha = jnp.maximum(
            av * (p1a_ref[0] + p1a_ref[1] + xsa_ref[...]) + b1v[:, :hh], 0.0)
        hb = jnp.maximum(
            av * (p1b_ref[0] + p1b_ref[1] + xsb_ref[...]) + b1v[:, hh:], 0.0)
        hw = (jnp.dot(ha, w2[:hh, :], preferred_element_type=jnp.float32)
              + jnp.dot(hb, w2[hh:, :], preferred_element_type=jnp.float32))
        out_ref[...] = av * hw

    return pl.pallas_call(
        body,
        grid=(n // rb,),
        in_specs=[
            pl.BlockSpec((2, rb, hh), lambda i: (0, i, 0)),
            pl.BlockSpec((2, rb, hh), lambda i: (0, i, 0)),
            pl.BlockSpec((rb, hh), lambda i: (i, 0)),
            pl.BlockSpec((rb, hh), lambda i: (i, 0)),
            pl.BlockSpec((rb, 1), lambda i: (i, 0)),
            pl.BlockSpec((1, h), lambda i: (0, 0)),
            pl.BlockSpec((h, co), lambda i: (0, 0)),
        ],
        out_specs=pl.BlockSpec((rb, co), lambda i: (i, 0)),
        out_shape=jax.ShapeDtypeStruct((NP_PAD, co), jnp.float32),
    )(p1a, p1b, xsa, xsb, a, b1, W2)


def _tc_softmax(p2, h2s, a, b2, n):
    co = h2s.shape[1]
    rb = 1000

    def body(p2_ref, h_ref, a_ref, b2_ref, out_ref):
        z = (a_ref[...] * (p2_ref[0] + p2_ref[1] + h_ref[...])
             + b2_ref[...])
        z = z - jnp.max(z, axis=1, keepdims=True)
        e = jnp.exp(z)
        out_ref[...] = e / jnp.sum(e, axis=1, keepdims=True)

    return pl.pallas_call(
        body,
        grid=(n // rb,),
        in_specs=[
            pl.BlockSpec((2, rb, co), lambda i: (0, i, 0)),
            pl.BlockSpec((rb, co), lambda i: (i, 0)),
            pl.BlockSpec((rb, 1), lambda i: (i, 0)),
            pl.BlockSpec((1, co), lambda i: (0, 0)),
        ],
        out_specs=pl.BlockSpec((rb, co), lambda i: (i, 0)),
        out_shape=jax.ShapeDtypeStruct((n, co), jnp.float32),
    )(p2, h2s, a, b2)


def kernel(node_embeddings, adjacency_lists, W1, b1, W2, b2):
    n, _ = node_embeddings.shape
    e = adjacency_lists.shape[1]
    adj = adjacency_lists.astype(jnp.int32)
    src = adj[0]
    dst = adj[1]
    ep = -(-e // EB) * EB
    if ep != e:
        src = jnp.concatenate([src, jnp.zeros((ep - e,), jnp.int32)])
        dst = jnp.concatenate([dst, jnp.full((ep - e,), n, jnp.int32)])
    src_rows = src.reshape(-1, EB)
    dst_rows = dst.reshape(-1, EB)

    degp = _sc_degree(dst_rows)
    xsa, xsb, a = _tc_scale(node_embeddings, W1, degp)
    hh = W1.shape[1] // 2
    p1a = _sc_spmm(xsa, src_rows, dst_rows, hh)
    p1b = _sc_spmm(xsb, src_rows, dst_rows, hh)
    h2s = _tc_hidden(p1a, p1b, xsa, xsb, a, b1.reshape(1, -1), W2)
    p2 = _sc_spmm(h2s, src_rows, dst_rows, W2.shape[1])
    return _tc_softmax(p2, h2s, a, b2.reshape(1, -1), n)
